# Initial kernel scaffold; baseline (speedup 1.0000x reference)
import jax
import jax.numpy as jnp
from jax import lax
from jax.experimental import pallas as pl
from jax.experimental.pallas import tpu as pltpu
from jax.experimental.pallas import tpu_sc as plsc

DENSE = 16384
B = 4096
K = 64
NC = 2
NS = 16
L = 16
NW = NC * NS
ROWS_PER_W = B // NW
KCHUNKS = K // L


def _sc_body(ids_hbm, vals_hbm, out_hbm, ids_v, vals_v, buf0, buf1,
             sem0, sem1):
    wid = lax.axis_index("s") * NC + lax.axis_index("c")
    base = wid * ROWS_PER_W

    pltpu.sync_copy(ids_hbm.at[pl.ds(base, ROWS_PER_W)], ids_v)
    pltpu.sync_copy(vals_hbm.at[pl.ds(base, ROWS_PER_W)], vals_v)

    zeros_f = jnp.zeros((L,), jnp.float32)

    def _zero(j, _):
        buf0[pl.ds(j * L, L)] = zeros_f
        buf1[pl.ds(j * L, L)] = zeros_f
        return 0
    lax.fori_loop(0, DENSE // L, _zero, 0)

    def scatter_add_row(buf, row):
        for c in range(KCHUNKS):
            idx = ids_v[row, pl.ds(c * L, L)]
            v = vals_v[row, pl.ds(c * L, L)]
            plsc.addupdate_scatter(buf, [idx], v)

    def scatter_zero_row(buf, row):
        for c in range(KCHUNKS):
            idx = ids_v[row, pl.ds(c * L, L)]
            plsc.store_scatter(buf, [idx], zeros_f)

    bufs = (buf0, buf1)
    sems = (sem0, sem1)

    for b in range(2):
        scatter_add_row(bufs[b], b)
        pltpu.async_copy(bufs[b], out_hbm.at[base + b], sems[b])

    def step(i, _):
        for b in range(2):
            row = i + b
            pltpu.make_async_copy(bufs[b], out_hbm.at[base], sems[b]).wait()
            scatter_zero_row(bufs[b], row - 2)
            scatter_add_row(bufs[b], row)
            pltpu.async_copy(bufs[b], out_hbm.at[base + row], sems[b])
        return 0

    lax.fori_loop(2, ROWS_PER_W, step, 0, unroll=False)

    for b in range(2):
        pltpu.make_async_copy(bufs[b], out_hbm.at[base], sems[b]).wait()


@jax.jit
def _sparse2dense(spike_ids, spike_vals):
    mesh = plsc.VectorSubcoreMesh(
        core_axis_name="c", subcore_axis_name="s",
        num_cores=NC, num_subcores=NS)
    return pl.kernel(
        _sc_body,
        out_type=jax.ShapeDtypeStruct((B, DENSE), jnp.float32),
        mesh=mesh,
        scratch_types=[
            pltpu.VMEM((ROWS_PER_W, K), jnp.int32),
            pltpu.VMEM((ROWS_PER_W, K), jnp.float32),
            pltpu.VMEM((DENSE,), jnp.float32),
            pltpu.VMEM((DENSE,), jnp.float32),
            pltpu.SemaphoreType.DMA,
            pltpu.SemaphoreType.DMA,
        ],
    )(spike_ids, spike_vals)


def kernel(spike_ids, spike_vals):
    return _sparse2dense(spike_ids, spike_vals)


# speedup vs baseline: 7.3532x; 7.3532x over previous
import jax
import jax.numpy as jnp
from jax import lax
from jax.experimental import pallas as pl
from jax.experimental.pallas import tpu as pltpu
from jax.experimental.pallas import tpu_sc as plsc

DENSE = 16384
B = 4096
K = 64
NC = 2
NS = 16
L = 16
NW = NC * NS
ROWS_PER_W = B // NW
KCHUNKS = K // L


def _sc_body(ids_hbm, vals_hbm, out_hbm, ids_v, vals_v, buf0, buf1,
             sem0, sem1):
    wid = lax.axis_index("s") * NC + lax.axis_index("c")
    base = wid * ROWS_PER_W

    pltpu.sync_copy(ids_hbm.at[pl.ds(base, ROWS_PER_W)], ids_v)
    pltpu.sync_copy(vals_hbm.at[pl.ds(base, ROWS_PER_W)], vals_v)

    zeros_f = jnp.zeros((L,), jnp.float32)

    def _zero(j, _):
        buf0[pl.ds(j * L, L)] = zeros_f
        buf1[pl.ds(j * L, L)] = zeros_f
        return 0
    lax.fori_loop(0, DENSE // L, _zero, 0)

    def scatter_add_row(buf, row):
        for c in range(KCHUNKS):
            idx = ids_v[row, pl.ds(c * L, L)]
            v = vals_v[row, pl.ds(c * L, L)]
            plsc.addupdate_scatter(buf, [idx], v)

    def scatter_zero_row(buf, row):
        for c in range(KCHUNKS):
            idx = ids_v[row, pl.ds(c * L, L)]
            plsc.store_scatter(buf, [idx], zeros_f)

    bufs = (buf0, buf1)
    sems = (sem0, sem1)

    for b in range(2):
        scatter_add_row(bufs[b], b)
        pltpu.async_copy(bufs[b], out_hbm.at[base + b], sems[b])

    def step(i, _):
        for b in range(2):
            row = 2 * i + b
            pltpu.make_async_copy(bufs[b], out_hbm.at[base], sems[b]).wait()
            scatter_zero_row(bufs[b], row - 2)
            scatter_add_row(bufs[b], row)
            pltpu.async_copy(bufs[b], out_hbm.at[base + row], sems[b])
        return 0

    lax.fori_loop(1, ROWS_PER_W // 2, step, 0, unroll=False)

    for b in range(2):
        pltpu.make_async_copy(bufs[b], out_hbm.at[base], sems[b]).wait()


@jax.jit
def _sparse2dense(spike_ids, spike_vals):
    mesh = plsc.VectorSubcoreMesh(
        core_axis_name="c", subcore_axis_name="s",
        num_cores=NC, num_subcores=NS)
    return pl.kernel(
        _sc_body,
        out_type=jax.ShapeDtypeStruct((B, DENSE), jnp.float32),
        mesh=mesh,
        compiler_params=pltpu.CompilerParams(needs_layout_passes=False),
        scratch_types=[
            pltpu.VMEM((ROWS_PER_W, K), jnp.int32),
            pltpu.VMEM((ROWS_PER_W, K), jnp.float32),
            pltpu.VMEM((DENSE,), jnp.float32),
            pltpu.VMEM((DENSE,), jnp.float32),
            pltpu.SemaphoreType.DMA,
            pltpu.SemaphoreType.DMA,
        ],
    )(spike_ids, spike_vals)


def kernel(spike_ids, spike_vals):
    return _sparse2dense(spike_ids, spike_vals)
